# Initial kernel scaffold; baseline (speedup 1.0000x reference)
import jax
import jax.numpy as jnp
from jax.experimental import pallas as pl


def kernel(x, edge_index, graph_ids, W0, b0, W1, b1, W2, b2, Wd1, bd1, Wd2, bd2):
    raise NotImplementedError("write your pallas kernel here")


# speedup vs baseline: 5.7395x; 5.7395x over previous
import functools
import jax
import jax.numpy as jnp
from jax import lax
from jax.experimental import pallas as pl
from jax.experimental.pallas import tpu as pltpu
from jax.experimental.pallas import tpu_sc as plsc

N = 10000
E = 160000
D = 256
H = 256
B = 16
NEG_SLOPE = 0.01

NP = 10240
NT = 16
CHUNK = 125
NCHUNK = (E // NT) // CHUNK
ROWS_PER_TILE = NP // NT

_sc_mesh = plsc.VectorSubcoreMesh(core_axis_name="c", subcore_axis_name="s")


@jax.jit
def _sc_degrees(ei4):
  @functools.partial(
      pl.kernel,
      out_type=jax.ShapeDtypeStruct((2, NP), jnp.float32),
      mesh=_sc_mesh,
      scratch_types=[
          pltpu.VMEM((NCHUNK, CHUNK), jnp.int32),
          pltpu.VMEM((128,), jnp.float32),
          pltpu.VMEM((ROWS_PER_TILE,), jnp.float32),
          pltpu.VMEM_SHARED((NP,), jnp.float32),
          pltpu.SemaphoreType.DMA,
      ],
  )
  def k(ei_hbm, deg_hbm, idx_v, ones_v, zbuf, hist_sh, sem):
    c = lax.axis_index("c")
    s = lax.axis_index("s")

    @pl.loop(0, 128, step=16)
    def _(i):
      ones_v[pl.ds(i, 16)] = jnp.full((16,), 1.0, jnp.float32)

    @pl.loop(0, ROWS_PER_TILE, step=16)
    def _(i):
      zbuf[pl.ds(i, 16)] = jnp.zeros((16,), jnp.float32)

    pltpu.sync_copy(zbuf, hist_sh.at[pl.ds(s * ROWS_PER_TILE, ROWS_PER_TILE)])
    pltpu.sync_copy(ei_hbm.at[c, s], idx_v)
    plsc.subcore_barrier()

    @pl.loop(0, NCHUNK)
    def _(j):
      pltpu.sync_copy(ones_v.at[pl.ds(0, CHUNK)], hist_sh.at[idx_v.at[j]],
                      add=True)

    plsc.subcore_barrier()
    pltpu.sync_copy(hist_sh.at[pl.ds(s * ROWS_PER_TILE, ROWS_PER_TILE)],
                    deg_hbm.at[c, pl.ds(s * ROWS_PER_TILE, ROWS_PER_TILE)])

  return k(ei4)


@jax.jit
def _sc_scatter(h, src3, dst3):
  @functools.partial(
      pl.kernel,
      out_type=jax.ShapeDtypeStruct((4, NP, 64), jnp.float32),
      mesh=_sc_mesh,
      scratch_types=[
          pltpu.VMEM((NCHUNK, CHUNK), jnp.int32),
          pltpu.VMEM((NCHUNK, CHUNK), jnp.int32),
          pltpu.VMEM((CHUNK, 64), jnp.float32),
          pltpu.VMEM((CHUNK, 64), jnp.float32),
          pltpu.VMEM((128, 64), jnp.float32),
          pltpu.VMEM_SHARED((NP, 64), jnp.float32),
          pltpu.SemaphoreType.DMA,
          pltpu.SemaphoreType.DMA,
      ],
      compiler_params=pltpu.CompilerParams(use_tc_tiling_on_sc=False),
  )
  def k(h_hbm, src_hbm, dst_hbm, agg_hbm,
        src_v, dst_v, buf0, buf1, zbuf, agg_sh, sem0, sem1):
    c = lax.axis_index("c")
    s = lax.axis_index("s")

    @pl.loop(0, 128)
    def _(r):
      @pl.loop(0, 64, step=16)
      def _(k2):
        zbuf[r, pl.ds(k2, 16)] = jnp.zeros((16,), jnp.float32)

    pltpu.sync_copy(src_hbm.at[s], src_v)
    pltpu.sync_copy(dst_hbm.at[s], dst_v)

    for p in range(2):
      slab = c * 2 + p

      @pl.loop(0, ROWS_PER_TILE // 128)
      def _(b):
        pltpu.sync_copy(
            zbuf, agg_sh.at[pl.ds(s * ROWS_PER_TILE + b * 128, 128)])

      plsc.subcore_barrier()

      def start(j, buf, sem):
        pltpu.async_copy(h_hbm.at[slab].at[src_v.at[j]], buf, sem)

      def wait(j, buf, sem):
        pltpu.make_async_copy(h_hbm.at[slab].at[src_v.at[j]], buf, sem).wait()

      start(0, buf0, sem0)
      start(1, buf1, sem1)

      @pl.loop(0, NCHUNK, step=2)
      def _(j):
        wait(j, buf0, sem0)
        pltpu.sync_copy(buf0, agg_sh.at[dst_v.at[j]], add=True)

        @pl.when(j + 2 < NCHUNK)
        def _():
          start(j + 2, buf0, sem0)

        wait(j + 1, buf1, sem1)
        pltpu.sync_copy(buf1, agg_sh.at[dst_v.at[j + 1]], add=True)

        @pl.when(j + 3 < NCHUNK)
        def _():
          start(j + 3, buf1, sem1)

      plsc.subcore_barrier()
      pltpu.sync_copy(agg_sh.at[pl.ds(s * ROWS_PER_TILE, ROWS_PER_TILE)],
                      agg_hbm.at[slab, pl.ds(s * ROWS_PER_TILE, ROWS_PER_TILE)])

  return k(h, src3, dst3)


BM = 1000
GRID = N // BM


def _slab_store(h_ref, h):
  for p in range(4):
    h_ref[p] = h[:, p * 64:(p + 1) * 64]


def _slab_concat(agg_ref):
  return jnp.concatenate([agg_ref[p] for p in range(4)], axis=1)


def _mm0_body(x_ref, od_ref, w_ref, h_ref):
  sc = lax.rsqrt(jnp.maximum(od_ref[...], 1.0))
  h = jnp.dot(x_ref[...] * sc, w_ref[...], preferred_element_type=jnp.float32)
  _slab_store(h_ref, h)


@jax.jit
def _tc_mm0(x, odeg, w0):
  return pl.pallas_call(
      _mm0_body,
      grid=(GRID,),
      in_specs=[
          pl.BlockSpec((BM, D), lambda i: (i, 0)),
          pl.BlockSpec((BM, 1), lambda i: (i, 0)),
          pl.BlockSpec((D, H), lambda i: (0, 0)),
      ],
      out_specs=pl.BlockSpec((4, BM, 64), lambda i: (0, i, 0)),
      out_shape=jax.ShapeDtypeStruct((4, N, 64), jnp.float32),
  )(x, odeg, w0)


def _pool_update(i, f, gid, rsum_ref, rmax_ref):
  onehot = (gid == lax.broadcasted_iota(jnp.int32, (1, B), 1)
            ).astype(jnp.float32)
  psum = lax.dot_general(onehot, f, (((0,), (0,)), ((), ())),
                         preferred_element_type=jnp.float32)

  @pl.when(i == 0)
  def _():
    rsum_ref[...] = psum
    rmax_ref[...] = jnp.full((B, H), -jnp.inf, jnp.float32)

  @pl.when(i > 0)
  def _():
    rsum_ref[...] += psum

  g_lo = gid[0, 0]
  g_hi = gid[BM - 1, 0]
  rows = lax.broadcasted_iota(jnp.int32, (B, 1), 0)

  def gbody(g, carry):
    m = jnp.max(jnp.where(gid == g, f, -jnp.inf), axis=0, keepdims=True)
    rmax_ref[...] = jnp.maximum(
        rmax_ref[...], jnp.where(rows == g, m, -jnp.inf))
    return carry

  lax.fori_loop(g_lo, g_hi + 1, gbody, 0)


def _tail_body(agg_ref, id_ref, od_ref, b_ref, gid_ref, wn_ref, prev_ref,
               hn_ref, merged_ref, rsum_ref, rmax_ref):
  i = pl.program_id(0)
  agg = _slab_concat(agg_ref)
  si = lax.rsqrt(jnp.maximum(id_ref[...], 1.0))
  f = agg * si + b_ref[...]
  f = jnp.where(f >= 0, f, NEG_SLOPE * f)
  _pool_update(i, f, gid_ref[...], rsum_ref, rmax_ref)

  so = lax.rsqrt(jnp.maximum(od_ref[...], 1.0))
  hn = jnp.dot(f * so, wn_ref[...], preferred_element_type=jnp.float32)
  _slab_store(hn_ref, hn)

  @pl.when(i == GRID - 1)
  def _():
    merged_ref[...] = prev_ref[...] + jnp.concatenate(
        [rsum_ref[...], rmax_ref[...]], axis=1)


@jax.jit
def _tc_tail(agg, ideg, odeg, bias, gid, wn, prev):
  return pl.pallas_call(
      _tail_body,
      grid=(GRID,),
      in_specs=[
          pl.BlockSpec((4, BM, 64), lambda i: (0, i, 0)),
          pl.BlockSpec((BM, 1), lambda i: (i, 0)),
          pl.BlockSpec((BM, 1), lambda i: (i, 0)),
          pl.BlockSpec((1, H), lambda i: (0, 0)),
          pl.BlockSpec((BM, 1), lambda i: (i, 0)),
          pl.BlockSpec((H, H), lambda i: (0, 0)),
          pl.BlockSpec((B, 2 * H), lambda i: (0, 0)),
      ],
      out_specs=[
          pl.BlockSpec((4, BM, 64), lambda i: (0, i, 0)),
          pl.BlockSpec((B, 2 * H), lambda i: (0, 0)),
      ],
      out_shape=[
          jax.ShapeDtypeStruct((4, N, 64), jnp.float32),
          jax.ShapeDtypeStruct((B, 2 * H), jnp.float32),
      ],
      scratch_shapes=[
          pltpu.VMEM((B, H), jnp.float32),
          pltpu.VMEM((B, H), jnp.float32),
      ],
  )(agg, ideg, odeg, bias, gid, wn, prev)


def _tail2_body(agg_ref, id_ref, b_ref, gid_ref, prev_ref,
                wd1_ref, bd1_ref, wd2_ref, bd2_ref,
                out_ref, rsum_ref, rmax_ref):
  i = pl.program_id(0)
  agg = _slab_concat(agg_ref)
  si = lax.rsqrt(jnp.maximum(id_ref[...], 1.0))
  f = agg * si + b_ref[...]
  f = jnp.where(f >= 0, f, NEG_SLOPE * f)
  _pool_update(i, f, gid_ref[...], rsum_ref, rmax_ref)

  @pl.when(i == GRID - 1)
  def _():
    merged = prev_ref[...] + jnp.concatenate(
        [rsum_ref[...], rmax_ref[...]], axis=1)
    d1 = jnp.dot(merged, wd1_ref[...],
                 preferred_element_type=jnp.float32) + bd1_ref[...]
    d2 = jnp.dot(d1, wd2_ref[...],
                 preferred_element_type=jnp.float32) + bd2_ref[...]
    out_ref[...] = jax.nn.sigmoid(d2)


@jax.jit
def _tc_tail2(agg, ideg, bias, gid, prev, wd1, bd1, wd2p, bd2p):
  return pl.pallas_call(
      _tail2_body,
      grid=(GRID,),
      in_specs=[
          pl.BlockSpec((4, BM, 64), lambda i: (0, i, 0)),
          pl.BlockSpec((BM, 1), lambda i: (i, 0)),
          pl.BlockSpec((1, H), lambda i: (0, 0)),
          pl.BlockSpec((BM, 1), lambda i: (i, 0)),
          pl.BlockSpec((B, 2 * H), lambda i: (0, 0)),
          pl.BlockSpec((2 * H, 128), lambda i: (0, 0)),
          pl.BlockSpec((1, 128), lambda i: (0, 0)),
          pl.BlockSpec((128, 128), lambda i: (0, 0)),
          pl.BlockSpec((1, 128), lambda i: (0, 0)),
      ],
      out_specs=pl.BlockSpec((B, 128), lambda i: (0, 0)),
      out_shape=jax.ShapeDtypeStruct((B, 128), jnp.float32),
      scratch_shapes=[
          pltpu.VMEM((B, H), jnp.float32),
          pltpu.VMEM((B, H), jnp.float32),
      ],
  )(agg, ideg, bias, gid, prev, wd1, bd1, wd2p, bd2p)


def kernel(x, edge_index, graph_ids, W0, b0, W1, b1, W2, b2, Wd1, bd1,
           Wd2, bd2):
  ei4 = edge_index.reshape(2, NT, NCHUNK, CHUNK)
  src3 = ei4[0]
  dst3 = ei4[1]

  degs = _sc_degrees(ei4)
  odeg = degs[0, :N, None]
  ideg = degs[1, :N, None]
  gid = graph_ids[:, None]

  wd2p = jnp.zeros((128, 128), jnp.float32).at[:, :2].set(Wd2)
  bd2p = jnp.zeros((1, 128), jnp.float32).at[0, :2].set(bd2)
  bd1r = bd1[None, :]

  h = _tc_mm0(x, odeg, W0)
  agg = _sc_scatter(h, src3, dst3)
  m = jnp.zeros((B, 2 * H), jnp.float32)
  h, m = _tc_tail(agg, ideg, odeg, b0[None, :], gid, W1, m)
  agg = _sc_scatter(h, src3, dst3)
  h, m = _tc_tail(agg, ideg, odeg, b1[None, :], gid, W2, m)
  agg = _sc_scatter(h, src3, dst3)
  outp = _tc_tail2(agg, ideg, b2[None, :], gid, m, Wd1, bd1r, wd2p, bd2p)
  return outp[:, :2]


# speedup vs baseline: 6.4170x; 1.1180x over previous
import functools
import jax
import jax.numpy as jnp
from jax import lax
from jax.experimental import pallas as pl
from jax.experimental.pallas import tpu as pltpu
from jax.experimental.pallas import tpu_sc as plsc

N = 10000
E = 160000
D = 256
H = 256
B = 16
NEG_SLOPE = 0.01

NP = 10240
NT = 16
CHUNK = 125
NCHUNK = (E // NT) // CHUNK
ROWS_PER_TILE = NP // NT

_sc_mesh = plsc.VectorSubcoreMesh(core_axis_name="c", subcore_axis_name="s")


@jax.jit
def _sc_degrees(ei4):
  @functools.partial(
      pl.kernel,
      out_type=jax.ShapeDtypeStruct((2, NP), jnp.float32),
      mesh=_sc_mesh,
      scratch_types=[
          pltpu.VMEM((NCHUNK, CHUNK), jnp.int32),
          pltpu.VMEM((128,), jnp.float32),
          pltpu.VMEM((ROWS_PER_TILE,), jnp.float32),
          pltpu.VMEM_SHARED((NP,), jnp.float32),
          pltpu.SemaphoreType.DMA,
      ],
  )
  def k(ei_hbm, deg_hbm, idx_v, ones_v, zbuf, hist_sh, sem):
    c = lax.axis_index("c")
    s = lax.axis_index("s")

    @pl.loop(0, 128, step=16)
    def _(i):
      ones_v[pl.ds(i, 16)] = jnp.full((16,), 1.0, jnp.float32)

    @pl.loop(0, ROWS_PER_TILE, step=16)
    def _(i):
      zbuf[pl.ds(i, 16)] = jnp.zeros((16,), jnp.float32)

    pltpu.sync_copy(zbuf, hist_sh.at[pl.ds(s * ROWS_PER_TILE, ROWS_PER_TILE)])
    pltpu.sync_copy(ei_hbm.at[c, s], idx_v)
    plsc.subcore_barrier()

    @pl.loop(0, NCHUNK)
    def _(j):
      pltpu.sync_copy(ones_v.at[pl.ds(0, CHUNK)], hist_sh.at[idx_v.at[j]],
                      add=True)

    plsc.subcore_barrier()
    pltpu.sync_copy(hist_sh.at[pl.ds(s * ROWS_PER_TILE, ROWS_PER_TILE)],
                    deg_hbm.at[c, pl.ds(s * ROWS_PER_TILE, ROWS_PER_TILE)])

  return k(ei4)


@jax.jit
def _sc_scatter(h, src3, dst3):
  @functools.partial(
      pl.kernel,
      out_type=jax.ShapeDtypeStruct((4, NP, 64), jnp.float32),
      mesh=_sc_mesh,
      scratch_types=[
          pltpu.VMEM((NCHUNK, CHUNK), jnp.int32),
          pltpu.VMEM((NCHUNK, CHUNK), jnp.int32),
          pltpu.VMEM((CHUNK, 64), jnp.float32),
          pltpu.VMEM((CHUNK, 64), jnp.float32),
          pltpu.VMEM((CHUNK, 64), jnp.float32),
          pltpu.VMEM((CHUNK, 64), jnp.float32),
          pltpu.VMEM((128, 64), jnp.float32),
          pltpu.VMEM_SHARED((NP, 64), jnp.float32),
          pltpu.SemaphoreType.DMA,
          pltpu.SemaphoreType.DMA,
          pltpu.SemaphoreType.DMA,
          pltpu.SemaphoreType.DMA,
          pltpu.SemaphoreType.DMA,
          pltpu.SemaphoreType.DMA,
          pltpu.SemaphoreType.DMA,
          pltpu.SemaphoreType.DMA,
      ],
      compiler_params=pltpu.CompilerParams(use_tc_tiling_on_sc=False),
  )
  def k(h_hbm, src_hbm, dst_hbm, agg_hbm,
        src_v, dst_v, buf0, buf1, buf2, buf3, zbuf, agg_sh,
        g0, g1, g2, g3, s0, s1, s2, s3):
    c = lax.axis_index("c")
    s = lax.axis_index("s")
    bufs = (buf0, buf1, buf2, buf3)
    gsems = (g0, g1, g2, g3)
    ssems = (s0, s1, s2, s3)

    @pl.loop(0, 128)
    def _(r):
      @pl.loop(0, 64, step=16)
      def _(k2):
        zbuf[r, pl.ds(k2, 16)] = jnp.zeros((16,), jnp.float32)

    pltpu.sync_copy(src_hbm.at[s], src_v)
    pltpu.sync_copy(dst_hbm.at[s], dst_v)

    for p in range(2):
      slab = c * 2 + p

      def g_start(ch, t):
        pltpu.async_copy(h_hbm.at[slab].at[src_v.at[ch]], bufs[t], gsems[t])

      def g_wait(ch, t):
        pltpu.make_async_copy(
            h_hbm.at[slab].at[src_v.at[ch]], bufs[t], gsems[t]).wait()

      def s_start(ch, t):
        pltpu.async_copy(bufs[t], agg_sh.at[dst_v.at[ch]], ssems[t], add=True)

      def s_wait(ch, t):
        pltpu.make_async_copy(
            bufs[t], agg_sh.at[dst_v.at[ch]], ssems[t]).wait()

      @pl.loop(0, ROWS_PER_TILE // 128)
      def _(b):
        pltpu.sync_copy(
            zbuf, agg_sh.at[pl.ds(s * ROWS_PER_TILE + b * 128, 128)])

      for t in range(3):
        g_start(t, t)

      plsc.subcore_barrier()

      @pl.loop(0, NCHUNK, step=4)
      def _(j):
        for t in range(4):
          ch = j + t
          g_wait(ch, t)
          s_start(ch, t)
          nt = (t + 3) % 4

          @pl.when(ch >= 1)
          def _():
            s_wait(ch - 1, nt)

          @pl.when(ch + 3 < NCHUNK)
          def _():
            g_start(ch + 3, nt)

      s_wait(NCHUNK - 1, (NCHUNK - 1) % 4)
      plsc.subcore_barrier()
      pltpu.sync_copy(agg_sh.at[pl.ds(s * ROWS_PER_TILE, ROWS_PER_TILE)],
                      agg_hbm.at[slab, pl.ds(s * ROWS_PER_TILE, ROWS_PER_TILE)])

  return k(h, src3, dst3)


BM = 1000
GRID = N // BM


def _slab_store(h_ref, h):
  for p in range(4):
    h_ref[p] = h[:, p * 64:(p + 1) * 64]


def _slab_concat(agg_ref):
  return jnp.concatenate([agg_ref[p] for p in range(4)], axis=1)


def _mm0_body(x_ref, od_ref, w_ref, h_ref):
  sc = lax.rsqrt(jnp.maximum(od_ref[...], 1.0))
  h = jnp.dot(x_ref[...] * sc, w_ref[...], preferred_element_type=jnp.float32)
  _slab_store(h_ref, h)


@jax.jit
def _tc_mm0(x, odeg, w0):
  return pl.pallas_call(
      _mm0_body,
      grid=(GRID,),
      in_specs=[
          pl.BlockSpec((BM, D), lambda i: (i, 0)),
          pl.BlockSpec((BM, 1), lambda i: (i, 0)),
          pl.BlockSpec((D, H), lambda i: (0, 0)),
      ],
      out_specs=pl.BlockSpec((4, BM, 64), lambda i: (0, i, 0)),
      out_shape=jax.ShapeDtypeStruct((4, N, 64), jnp.float32),
  )(x, odeg, w0)


def _pool_update(i, f, gid, rsum_ref, rmax_ref):
  onehot = (gid == lax.broadcasted_iota(jnp.int32, (1, B), 1)
            ).astype(jnp.float32)
  psum = lax.dot_general(onehot, f, (((0,), (0,)), ((), ())),
                         preferred_element_type=jnp.float32)

  @pl.when(i == 0)
  def _():
    rsum_ref[...] = psum
    rmax_ref[...] = jnp.full((B, H), -jnp.inf, jnp.float32)

  @pl.when(i > 0)
  def _():
    rsum_ref[...] += psum

  g_lo = gid[0, 0]
  g_hi = gid[BM - 1, 0]
  rows = lax.broadcasted_iota(jnp.int32, (B, 1), 0)

  def gbody(g, carry):
    m = jnp.max(jnp.where(gid == g, f, -jnp.inf), axis=0, keepdims=True)
    rmax_ref[...] = jnp.maximum(
        rmax_ref[...], jnp.where(rows == g, m, -jnp.inf))
    return carry

  lax.fori_loop(g_lo, g_hi + 1, gbody, 0)


def _tail_body(agg_ref, id_ref, od_ref, b_ref, gid_ref, wn_ref, prev_ref,
               hn_ref, merged_ref, rsum_ref, rmax_ref):
  i = pl.program_id(0)
  agg = _slab_concat(agg_ref)
  si = lax.rsqrt(jnp.maximum(id_ref[...], 1.0))
  f = agg * si + b_ref[...]
  f = jnp.where(f >= 0, f, NEG_SLOPE * f)
  _pool_update(i, f, gid_ref[...], rsum_ref, rmax_ref)

  so = lax.rsqrt(jnp.maximum(od_ref[...], 1.0))
  hn = jnp.dot(f * so, wn_ref[...], preferred_element_type=jnp.float32)
  _slab_store(hn_ref, hn)

  @pl.when(i == GRID - 1)
  def _():
    merged_ref[...] = prev_ref[...] + jnp.concatenate(
        [rsum_ref[...], rmax_ref[...]], axis=1)


@jax.jit
def _tc_tail(agg, ideg, odeg, bias, gid, wn, prev):
  return pl.pallas_call(
      _tail_body,
      grid=(GRID,),
      in_specs=[
          pl.BlockSpec((4, BM, 64), lambda i: (0, i, 0)),
          pl.BlockSpec((BM, 1), lambda i: (i, 0)),
          pl.BlockSpec((BM, 1), lambda i: (i, 0)),
          pl.BlockSpec((1, H), lambda i: (0, 0)),
          pl.BlockSpec((BM, 1), lambda i: (i, 0)),
          pl.BlockSpec((H, H), lambda i: (0, 0)),
          pl.BlockSpec((B, 2 * H), lambda i: (0, 0)),
      ],
      out_specs=[
          pl.BlockSpec((4, BM, 64), lambda i: (0, i, 0)),
          pl.BlockSpec((B, 2 * H), lambda i: (0, 0)),
      ],
      out_shape=[
          jax.ShapeDtypeStruct((4, N, 64), jnp.float32),
          jax.ShapeDtypeStruct((B, 2 * H), jnp.float32),
      ],
      scratch_shapes=[
          pltpu.VMEM((B, H), jnp.float32),
          pltpu.VMEM((B, H), jnp.float32),
      ],
  )(agg, ideg, odeg, bias, gid, wn, prev)


def _tail2_body(agg_ref, id_ref, b_ref, gid_ref, prev_ref,
                wd1_ref, bd1_ref, wd2_ref, bd2_ref,
                out_ref, rsum_ref, rmax_ref):
  i = pl.program_id(0)
  agg = _slab_concat(agg_ref)
  si = lax.rsqrt(jnp.maximum(id_ref[...], 1.0))
  f = agg * si + b_ref[...]
  f = jnp.where(f >= 0, f, NEG_SLOPE * f)
  _pool_update(i, f, gid_ref[...], rsum_ref, rmax_ref)

  @pl.when(i == GRID - 1)
  def _():
    merged = prev_ref[...] + jnp.concatenate(
        [rsum_ref[...], rmax_ref[...]], axis=1)
    d1 = jnp.dot(merged, wd1_ref[...],
                 preferred_element_type=jnp.float32) + bd1_ref[...]
    d2 = jnp.dot(d1, wd2_ref[...],
                 preferred_element_type=jnp.float32) + bd2_ref[...]
    out_ref[...] = jax.nn.sigmoid(d2)


@jax.jit
def _tc_tail2(agg, ideg, bias, gid, prev, wd1, bd1, wd2p, bd2p):
  return pl.pallas_call(
      _tail2_body,
      grid=(GRID,),
      in_specs=[
          pl.BlockSpec((4, BM, 64), lambda i: (0, i, 0)),
          pl.BlockSpec((BM, 1), lambda i: (i, 0)),
          pl.BlockSpec((1, H), lambda i: (0, 0)),
          pl.BlockSpec((BM, 1), lambda i: (i, 0)),
          pl.BlockSpec((B, 2 * H), lambda i: (0, 0)),
          pl.BlockSpec((2 * H, 128), lambda i: (0, 0)),
          pl.BlockSpec((1, 128), lambda i: (0, 0)),
          pl.BlockSpec((128, 128), lambda i: (0, 0)),
          pl.BlockSpec((1, 128), lambda i: (0, 0)),
      ],
      out_specs=pl.BlockSpec((B, 128), lambda i: (0, 0)),
      out_shape=jax.ShapeDtypeStruct((B, 128), jnp.float32),
      scratch_shapes=[
          pltpu.VMEM((B, H), jnp.float32),
          pltpu.VMEM((B, H), jnp.float32),
      ],
  )(agg, ideg, bias, gid, prev, wd1, bd1, wd2p, bd2p)


def kernel(x, edge_index, graph_ids, W0, b0, W1, b1, W2, b2, Wd1, bd1,
           Wd2, bd2):
  ei4 = edge_index.reshape(2, NT, NCHUNK, CHUNK)
  src3 = ei4[0]
  dst3 = ei4[1]

  degs = _sc_degrees(ei4)
  odeg = degs[0, :N, None]
  ideg = degs[1, :N, None]
  gid = graph_ids[:, None]

  wd2p = jnp.zeros((128, 128), jnp.float32).at[:, :2].set(Wd2)
  bd2p = jnp.zeros((1, 128), jnp.float32).at[0, :2].set(bd2)
  bd1r = bd1[None, :]

  h = _tc_mm0(x, odeg, W0)
  agg = _sc_scatter(h, src3, dst3)
  m = jnp.zeros((B, 2 * H), jnp.float32)
  h, m = _tc_tail(agg, ideg, odeg, b0[None, :], gid, W1, m)
  agg = _sc_scatter(h, src3, dst3)
  h, m = _tc_tail(agg, ideg, odeg, b1[None, :], gid, W2, m)
  agg = _sc_scatter(h, src3, dst3)
  outp = _tc_tail2(agg, ideg, b2[None, :], gid, m, Wd1, bd1r, wd2p, bd2p)
  return outp[:, :2]
